# baseline (speedup 1.0000x reference)
import functools

import jax
import jax.numpy as jnp
from jax.experimental import pallas as pl

B, S, H, E, K, I = 1, 2048, 1024, 8, 2, 512
ALPHA, LIMIT = 1.702, 7.0


def _router_kernel(x_ref, w_ref, b_ref, scores_ref):
    x = x_ref[...]
    logits = jnp.dot(x, w_ref[...], preferred_element_type=jnp.float32)
    logits = logits + b_ref[...][None, :]
    t = logits.shape[0]
    eidx = jax.lax.broadcasted_iota(jnp.int32, (t, E), 1)
    m1 = jnp.max(logits, axis=1, keepdims=True)
    idx1 = jnp.min(jnp.where(logits == m1, eidx, E), axis=1, keepdims=True)
    masked = jnp.where(eidx == idx1, -jnp.inf, logits)
    m2 = jnp.max(masked, axis=1, keepdims=True)
    idx2 = jnp.min(jnp.where(masked == m2, eidx, E), axis=1, keepdims=True)
    b2 = jnp.exp(m2 - m1)
    w1 = 1.0 / (1.0 + b2)
    w2 = b2 / (1.0 + b2)
    scores = jnp.where(eidx == idx1, w1, 0.0) + jnp.where(eidx == idx2, w2, 0.0)
    scores_ref[...] = scores.astype(scores_ref.dtype)


def _expert_kernel(x_ref, scores_ref, gu_ref, gub_ref, dp_ref, db_ref, out_ref):
    e = pl.program_id(0)
    x = x_ref[...]
    g = jnp.dot(x, gu_ref[0], preferred_element_type=jnp.float32)
    g = g + gub_ref[0]
    gate = jnp.minimum(g[:, :I], LIMIT)
    up = jnp.clip(g[:, I:], -LIMIT, LIMIT)
    glu = gate * jax.nn.sigmoid(gate * ALPHA)
    act = (up + 1.0) * glu
    d = jnp.dot(act.astype(jnp.bfloat16), dp_ref[0],
                preferred_element_type=jnp.float32)
    sc = scores_ref[...]
    eidx = jax.lax.broadcasted_iota(jnp.int32, sc.shape, 1)
    s = jnp.sum(jnp.where(eidx == e, sc, 0.0), axis=1, keepdims=True)

    @pl.when(e == 0)
    def _():
        out_ref[...] = db_ref[...][None, :] + s * d

    @pl.when(e != 0)
    def _():
        out_ref[...] += s * d


def kernel(hidden_states, router_weight, router_bias, gate_up_proj,
           gate_up_proj_bias, down_proj, down_proj_bias):
    b, s, h = hidden_states.shape
    x2d = hidden_states.reshape(b * s, h)
    T = b * s

    scores = pl.pallas_call(
        _router_kernel,
        grid=(T // 256,),
        in_specs=[
            pl.BlockSpec((256, H), lambda i: (i, 0)),
            pl.BlockSpec((H, E), lambda i: (0, 0)),
            pl.BlockSpec((E,), lambda i: (0,)),
        ],
        out_specs=pl.BlockSpec((256, E), lambda i: (i, 0)),
        out_shape=jax.ShapeDtypeStruct((T, E), hidden_states.dtype),
    )(x2d, router_weight, router_bias)

    gu = gate_up_proj.reshape(E, H, 2 * I).astype(jnp.bfloat16)
    dp = down_proj.reshape(E, I, H).astype(jnp.bfloat16)
    xb = x2d.astype(jnp.bfloat16)

    out = pl.pallas_call(
        _expert_kernel,
        grid=(E,),
        in_specs=[
            pl.BlockSpec((T, H), lambda e: (0, 0)),
            pl.BlockSpec((T, E), lambda e: (0, 0)),
            pl.BlockSpec((1, H, 2 * I), lambda e: (e, 0, 0)),
            pl.BlockSpec((1, 1, 2 * I), lambda e: (e, 0, 0)),
            pl.BlockSpec((1, I, H), lambda e: (e, 0, 0)),
            pl.BlockSpec((H,), lambda e: (0,)),
        ],
        out_specs=pl.BlockSpec((T, H), lambda e: (0, 0)),
        out_shape=jax.ShapeDtypeStruct((T, H), jnp.float32),
    )(xb, scores, gu, gate_up_proj_bias.reshape(E, 1, 2 * I), dp, down_proj_bias)

    return (out.reshape(b, s, h), scores)


# speedup vs baseline: 1.3763x; 1.3763x over previous
import jax
import jax.numpy as jnp
from jax.experimental import pallas as pl
from jax.experimental.pallas import tpu as pltpu

B, S, H, E, K, I = 1, 2048, 1024, 8, 2, 512
ALPHA, LIMIT = 1.702, 7.0


def _moe_kernel(x_ref, w_ref, b_ref, gu_ref, gub_ref, dp_ref, db_ref,
                scores_ref, out_ref, xb_ref, acc_ref):
    j = pl.program_id(0)

    @pl.when(j == 0)
    def _():
        x = x_ref[...]
        logits = jnp.dot(x, w_ref[...], preferred_element_type=jnp.float32)
        logits = logits + b_ref[...][None, :]
        eidx = jax.lax.broadcasted_iota(jnp.int32, (S, E), 1)
        m1 = jnp.max(logits, axis=1, keepdims=True)
        idx1 = jnp.min(jnp.where(logits == m1, eidx, E), axis=1, keepdims=True)
        masked = jnp.where(eidx == idx1, -jnp.inf, logits)
        m2 = jnp.max(masked, axis=1, keepdims=True)
        idx2 = jnp.min(jnp.where(masked == m2, eidx, E), axis=1, keepdims=True)
        b2 = jnp.exp(m2 - m1)
        w1 = 1.0 / (1.0 + b2)
        w2 = b2 / (1.0 + b2)
        scores_ref[...] = (jnp.where(eidx == idx1, w1, 0.0)
                           + jnp.where(eidx == idx2, w2, 0.0))
        xb_ref[...] = x.astype(jnp.bfloat16)

    @pl.when(j > 0)
    def _():
        e = j - 1
        xb = xb_ref[...]
        g = jnp.dot(xb, gu_ref[0].astype(jnp.bfloat16),
                    preferred_element_type=jnp.float32)
        g = g + gub_ref[0]
        gate = jnp.minimum(g[:, :I], LIMIT)
        up = jnp.clip(g[:, I:], -LIMIT, LIMIT)
        glu = gate * jax.nn.sigmoid(gate * ALPHA)
        act = (up + 1.0) * glu
        d = jnp.dot(act.astype(jnp.bfloat16), dp_ref[0].astype(jnp.bfloat16),
                    preferred_element_type=jnp.float32)
        sc = scores_ref[...]
        eidx = jax.lax.broadcasted_iota(jnp.int32, (S, E), 1)
        s_col = jnp.sum(jnp.where(eidx == e, sc, 0.0), axis=1, keepdims=True)
        wd = s_col * d

        @pl.when(j == 1)
        def _():
            acc_ref[...] = wd.astype(jnp.bfloat16)

        @pl.when((j > 1) & (j < E))
        def _():
            acc_ref[...] = (acc_ref[...].astype(jnp.float32)
                            + wd).astype(jnp.bfloat16)

        @pl.when(j == E)
        def _():
            out_ref[...] = (acc_ref[...].astype(jnp.float32) + wd
                            + db_ref[...][None, :])


def kernel(hidden_states, router_weight, router_bias, gate_up_proj,
           gate_up_proj_bias, down_proj, down_proj_bias):
    b, s, h = hidden_states.shape
    x2d = hidden_states.reshape(S, H)
    gu = gate_up_proj.reshape(E, H, 2 * I)
    dp = down_proj.reshape(E, I, H)

    scores, out = pl.pallas_call(
        _moe_kernel,
        grid=(E + 1,),
        in_specs=[
            pl.BlockSpec((S, H), lambda j: (0, 0)),
            pl.BlockSpec((H, E), lambda j: (0, 0)),
            pl.BlockSpec((E,), lambda j: (0,)),
            pl.BlockSpec((1, H, 2 * I), lambda j: (jnp.maximum(j - 1, 0), 0, 0)),
            pl.BlockSpec((1, 1, 2 * I), lambda j: (jnp.maximum(j - 1, 0), 0, 0)),
            pl.BlockSpec((1, I, H), lambda j: (jnp.maximum(j - 1, 0), 0, 0)),
            pl.BlockSpec((H,), lambda j: (0,)),
        ],
        out_specs=[
            pl.BlockSpec((S, E), lambda j: (0, 0)),
            pl.BlockSpec((S, H), lambda j: (0, 0)),
        ],
        out_shape=[
            jax.ShapeDtypeStruct((S, E), jnp.float32),
            jax.ShapeDtypeStruct((S, H), jnp.float32),
        ],
        scratch_shapes=[
            pltpu.VMEM((S, H), jnp.bfloat16),
            pltpu.VMEM((S, H), jnp.bfloat16),
        ],
    )(x2d, router_weight, router_bias, gu,
      gate_up_proj_bias.reshape(E, 1, 2 * I), dp, down_proj_bias)

    return (out.reshape(b, s, h), scores)
